# Initial kernel scaffold; baseline (speedup 1.0000x reference)
import jax
import jax.numpy as jnp
from jax.experimental import pallas as pl


def kernel(x, table):
    raise NotImplementedError("write your pallas kernel here")


# speedup vs baseline: 1.1813x; 1.1813x over previous
import jax
import jax.numpy as jnp
from jax import lax
from jax.experimental import pallas as pl
from jax.experimental.pallas import tpu as pltpu
from jax.experimental.pallas import tpu_sc as plsc

NC = 2
NS = 16
NW = NC * NS

D = 32
ROWS = 128
J = 8
GROUP = J * ROWS


def _sc_gather(x_flat, table, ngroups):
    mesh = plsc.VectorSubcoreMesh(core_axis_name="c", subcore_axis_name="s")

    @pl.kernel(
        out_type=jax.ShapeDtypeStruct((NW, ngroups, J, ROWS, D), jnp.float32),
        mesh=mesh,
        compiler_params=pltpu.CompilerParams(use_tc_tiling_on_sc=False),
        scratch_types=[
            pltpu.VMEM((J, ROWS), jnp.int32),
            pltpu.VMEM((ROWS, D), jnp.float32),
            pltpu.SemaphoreType.DMA,
        ],
    )
    def k(x_hbm, table_hbm, out_hbm, idx_v, rows_v, sem):
        cid = lax.axis_index("c")
        sid = lax.axis_index("s")
        wid = sid * NC + cid

        def group(g, carry):
            pltpu.sync_copy(x_hbm.at[wid, g], idx_v)
            for j in range(J):
                pltpu.async_copy(table_hbm.at[idx_v.at[j]], rows_v, sem).wait()
                pltpu.sync_copy(rows_v, out_hbm.at[wid, g, j])
            return carry

        lax.fori_loop(0, ngroups, group, 0)

    return k(x_flat, table)


def kernel(x, table):
    n = x.shape[0] * x.shape[1]
    ngroups = n // (NW * GROUP)
    x_flat = x.reshape(NW, ngroups, J, ROWS).astype(jnp.int32)
    out = _sc_gather(x_flat, table, ngroups)
    return out.reshape(x.shape[0], x.shape[1], D)


# speedup vs baseline: 1.3093x; 1.1083x over previous
import jax
import jax.numpy as jnp
from jax import lax
from jax.experimental import pallas as pl
from jax.experimental.pallas import tpu as pltpu
from jax.experimental.pallas import tpu_sc as plsc

NC = 2
NS = 16
NW = NC * NS

D = 32
ROWS = 128
NBUF = 8


def _sc_gather(x_flat, table, nchunk):
    mesh = plsc.VectorSubcoreMesh(core_axis_name="c", subcore_axis_name="s")
    nrounds = nchunk // NBUF

    @pl.kernel(
        out_type=jax.ShapeDtypeStruct((NW, nchunk, ROWS, D), jnp.float32),
        mesh=mesh,
        compiler_params=pltpu.CompilerParams(use_tc_tiling_on_sc=False),
        scratch_types=[
            pltpu.VMEM((nchunk, ROWS), jnp.int32),
            pltpu.VMEM((NBUF, ROWS, D), jnp.float32),
            pltpu.SemaphoreType.DMA((NBUF,)),
            pltpu.SemaphoreType.DMA((NBUF,)),
        ],
    )
    def k(x_hbm, table_hbm, out_hbm, idx_v, rows_v, gsem, osem):
        cid = lax.axis_index("c")
        sid = lax.axis_index("s")
        wid = sid * NC + cid

        pltpu.sync_copy(x_hbm.at[wid], idx_v)

        def fire(r, b, wait_out):
            c = r * NBUF + b
            if wait_out:
                pltpu.make_async_copy(
                    rows_v.at[b], out_hbm.at[wid, c - NBUF], osem.at[b]
                ).wait()
            pltpu.async_copy(table_hbm.at[idx_v.at[c]], rows_v.at[b],
                             gsem.at[b])

        def drain(r, b):
            c = r * NBUF + b
            pltpu.make_async_copy(
                table_hbm.at[idx_v.at[c]], rows_v.at[b], gsem.at[b]
            ).wait()
            pltpu.async_copy(rows_v.at[b], out_hbm.at[wid, c], osem.at[b])

        for b in range(NBUF):
            fire(0, b, wait_out=False)
        for b in range(NBUF):
            drain(0, b)

        def round_body(r, carry):
            for b in range(NBUF):
                fire(r, b, wait_out=True)
            for b in range(NBUF):
                drain(r, b)
            return carry

        lax.fori_loop(1, nrounds, round_body, 0)

        for b in range(NBUF):
            c = (nrounds - 1) * NBUF + b
            pltpu.make_async_copy(
                rows_v.at[b], out_hbm.at[wid, c], osem.at[b]
            ).wait()

    return k(x_flat, table)


def kernel(x, table):
    n = x.shape[0] * x.shape[1]
    nchunk = n // (NW * ROWS)
    x_flat = x.reshape(NW, nchunk, ROWS).astype(jnp.int32)
    out = _sc_gather(x_flat, table, nchunk)
    return out.reshape(x.shape[0], x.shape[1], D)


# speedup vs baseline: 1.7727x; 1.3540x over previous
import jax
import jax.numpy as jnp
from jax import lax
from jax.experimental import pallas as pl
from jax.experimental.pallas import tpu as pltpu
from jax.experimental.pallas import tpu_sc as plsc

NC = 2
NS = 16
NW = NC * NS

D = 32
XR = 1
NBUF = 8


def _sc_gather(x, table):
    N, S = x.shape
    rpw = N // NW
    gpw = rpw // XR
    nrounds = gpw // NBUF
    mesh = plsc.VectorSubcoreMesh(core_axis_name="c", subcore_axis_name="s")

    @pl.kernel(
        out_type=jax.ShapeDtypeStruct((N, S, D), jnp.float32),
        mesh=mesh,
        compiler_params=pltpu.CompilerParams(use_tc_tiling_on_sc=False),
        scratch_types=[
            pltpu.VMEM((rpw, S), jnp.int32),
            pltpu.VMEM((NBUF, S, D), jnp.float32),
            pltpu.SemaphoreType.DMA((NBUF,)),
            pltpu.SemaphoreType.DMA((NBUF,)),
        ],
    )
    def k(x_hbm, table_hbm, out_hbm, idx_v, rows_v, gsem, osem):
        cid = lax.axis_index("c")
        sid = lax.axis_index("s")
        wid = sid * NC + cid
        base = wid * rpw

        pltpu.sync_copy(x_hbm.at[pl.ds(base, rpw)], idx_v)

        def fire(r, b, wait_out):
            g = r * NBUF + b
            if wait_out:
                pltpu.make_async_copy(
                    rows_v.at[b],
                    out_hbm.at[base + g - NBUF],
                    osem.at[b],
                ).wait()
            pltpu.async_copy(
                table_hbm.at[idx_v.at[g]],
                rows_v.at[b],
                gsem.at[b],
            )

        def drain(r, b):
            g = r * NBUF + b
            pltpu.make_async_copy(
                table_hbm.at[idx_v.at[g]],
                rows_v.at[b],
                gsem.at[b],
            ).wait()
            pltpu.async_copy(
                rows_v.at[b],
                out_hbm.at[base + g],
                osem.at[b],
            )

        for b in range(NBUF):
            fire(0, b, wait_out=False)
        for b in range(NBUF):
            drain(0, b)

        def round_body(r, carry):
            for b in range(NBUF):
                fire(r, b, wait_out=True)
            for b in range(NBUF):
                drain(r, b)
            return carry

        lax.fori_loop(1, nrounds, round_body, 0)

        for b in range(NBUF):
            g = (nrounds - 1) * NBUF + b
            pltpu.make_async_copy(
                rows_v.at[b],
                out_hbm.at[base + g],
                osem.at[b],
            ).wait()

    return k(x, table)


def kernel(x, table):
    return _sc_gather(x.astype(jnp.int32), table)
